# baseline (speedup 1.0000x reference)
import functools

import jax
import jax.numpy as jnp
from jax import lax
from jax.experimental import pallas as pl
from jax.experimental.pallas import tpu as pltpu
from jax.experimental.pallas import tpu_sc as plsc

D = 16
BATCH = 16384
NW = 32
BPW = BATCH // NW
CHUNK = 128
NCH = BPW // CHUNK
COFF = 0.1

_mesh = plsc.VectorSubcoreMesh(core_axis_name="c", subcore_axis_name="s")


@functools.partial(
    pl.kernel,
    out_type=jax.ShapeDtypeStruct((5, BATCH, D), jnp.float32),
    mesh=_mesh,
    compiler_params=pltpu.CompilerParams(use_tc_tiling_on_sc=False),
    scratch_types=[
        pltpu.VMEM((BPW,), jnp.int32),
        pltpu.VMEM((BPW, D), jnp.float32),
        pltpu.SemaphoreType.DMA,
    ],
)
def _gather5(idx_hbm, table_hbm, out_hbm, idx_v, rows_v, sem):
    wid = lax.axis_index("s") * 2 + lax.axis_index("c")
    base = wid * BPW
    for j in range(5):
        pltpu.sync_copy(idx_hbm.at[pl.ds(j * BATCH + base, BPW)], idx_v)
        copies = []
        for k in range(NCH):
            copies.append(
                pltpu.async_copy(
                    table_hbm.at[idx_v.at[pl.ds(k * CHUNK, CHUNK)]],
                    rows_v.at[pl.ds(k * CHUNK, CHUNK)],
                    sem,
                )
            )
        for c in copies:
            c.wait()
        pltpu.sync_copy(rows_v, out_hbm.at[j, pl.ds(base, BPW)])


def _softplus(z):
    return jnp.maximum(z, 0.0) + jnp.log1p(jnp.exp(-jnp.abs(z)))


def _loss_body(g_ref, out_ref):
    ru = g_ref[0]
    rpi = g_ref[1]
    rni = g_ref[2]
    rpe = g_ref[3]
    rne = g_ref[4]

    d = lax.broadcasted_iota(jnp.int32, (128, 8), 0)
    k = lax.broadcasted_iota(jnp.int32, (128, 8), 1)
    sel = jnp.where(d // D == k, 1.0, 0.0).astype(jnp.float32)

    def gsum(x):
        return jnp.dot(x, sel, preferred_element_type=jnp.float32)

    pos_pred = gsum(ru * rpi)
    neg_pred = gsum(ru * rni)
    cf = jnp.sum(_softplus(neg_pred - pos_pred))

    n_pi = gsum(rpi * rpi)
    n_pe = gsum(rpe * rpe)
    n_ne = gsum(rne * rne)
    a = gsum(rpi * rpe)
    b = gsum(rpi * rne)
    iv_pi = 1.0 / jnp.maximum(jnp.sqrt(n_pi), 1e-12)
    iv_pe = 1.0 / jnp.maximum(jnp.sqrt(n_pe), 1e-12)
    iv_ne = 1.0 / jnp.maximum(jnp.sqrt(n_ne), 1e-12)
    pos_reg = n_pi * iv_pi * iv_pi - 2.0 * a * iv_pi * iv_pe + n_pe * iv_pe * iv_pe
    neg_reg = n_pi * iv_pi * iv_pi - 2.0 * b * iv_pi * iv_ne + n_ne * iv_ne * iv_ne
    reg = jnp.sum(_softplus(neg_reg - pos_reg))

    out_ref[0, 0] = cf + COFF * reg


_loss = pl.pallas_call(
    _loss_body,
    out_shape=jax.ShapeDtypeStruct((1, 1), jnp.float32),
    in_specs=[pl.BlockSpec(memory_space=pltpu.VMEM)],
    out_specs=pl.BlockSpec(memory_space=pltpu.SMEM),
)


@jax.jit
def kernel(repr_x, pos_neg_pair_t):
    idx = pos_neg_pair_t.T.reshape(5 * BATCH)
    g = _gather5(idx, repr_x)
    g = g.reshape(5, BATCH * D // 128, 128)
    return _loss(g)[0, 0]


# speedup vs baseline: 1.0037x; 1.0037x over previous
import functools

import jax
import jax.numpy as jnp
from jax import lax
from jax.experimental import pallas as pl
from jax.experimental.pallas import tpu as pltpu
from jax.experimental.pallas import tpu_sc as plsc

D = 16
BATCH = 16384
NW = 32
BPW = BATCH // NW
CHUNK = 128
NCH = BPW // CHUNK
COFF = 0.1
R = BATCH * D // 128

_mesh = plsc.VectorSubcoreMesh(core_axis_name="c", subcore_axis_name="s")


@functools.partial(
    pl.kernel,
    out_type=jax.ShapeDtypeStruct((5, BATCH, D), jnp.float32),
    mesh=_mesh,
    compiler_params=pltpu.CompilerParams(
        use_tc_tiling_on_sc=False, needs_layout_passes=False
    ),
    scratch_types=[
        pltpu.VMEM((5 * BPW,), jnp.int32),
        pltpu.VMEM((5 * BPW,), jnp.int32),
        pltpu.VMEM((5, BPW, D), jnp.float32),
        pltpu.SemaphoreType.DMA,
    ],
)
def _gather5(pnp_hbm, table_hbm, out_hbm, pnp_v, idx_v, rows_v, sem):
    wid = lax.axis_index("s") * 2 + lax.axis_index("c")
    base = wid * BPW
    pltpu.sync_copy(pnp_hbm.at[pl.ds(base * 5, 5 * BPW)], pnp_v)
    lane = lax.iota(jnp.int32, 16)
    l5 = lane * 5
    for j in range(5):
        for g in range(BPW // 16):
            vals = plsc.load_gather(pnp_v, [l5 + (g * 80 + j)])
            idx_v[pl.ds(j * BPW + g * 16, 16)] = vals
    copies = []
    for j in range(5):
        for k in range(NCH):
            copies.append(
                pltpu.async_copy(
                    table_hbm.at[idx_v.at[pl.ds(j * BPW + k * CHUNK, CHUNK)]],
                    rows_v.at[j, pl.ds(k * CHUNK, CHUNK), :],
                    sem,
                )
            )
    for c in copies:
        c.wait()
    for j in range(5):
        pltpu.sync_copy(rows_v.at[j], out_hbm.at[j, pl.ds(base, BPW), :])


def _softplus(z):
    return jnp.maximum(z, 0.0) + jnp.log1p(jnp.exp(-jnp.abs(z)))


def _loss_body(g_ref, out_ref):
    ru = g_ref[pl.ds(0 * R, R), :]
    rpi = g_ref[pl.ds(1 * R, R), :]
    rni = g_ref[pl.ds(2 * R, R), :]
    rpe = g_ref[pl.ds(3 * R, R), :]
    rne = g_ref[pl.ds(4 * R, R), :]

    d = lax.broadcasted_iota(jnp.int32, (128, 8), 0)
    k = lax.broadcasted_iota(jnp.int32, (128, 8), 1)
    sel = jnp.where(d // D == k, 1.0, 0.0).astype(jnp.float32)

    def gsum(x):
        return jnp.dot(x, sel, preferred_element_type=jnp.float32)

    pos_pred = gsum(ru * rpi)
    neg_pred = gsum(ru * rni)
    cf = jnp.sum(_softplus(neg_pred - pos_pred))

    n_pi = gsum(rpi * rpi)
    n_pe = gsum(rpe * rpe)
    n_ne = gsum(rne * rne)
    a = gsum(rpi * rpe)
    b = gsum(rpi * rne)
    iv_pi = 1.0 / jnp.maximum(jnp.sqrt(n_pi), 1e-12)
    iv_pe = 1.0 / jnp.maximum(jnp.sqrt(n_pe), 1e-12)
    iv_ne = 1.0 / jnp.maximum(jnp.sqrt(n_ne), 1e-12)
    pos_reg = n_pi * iv_pi * iv_pi - 2.0 * a * iv_pi * iv_pe + n_pe * iv_pe * iv_pe
    neg_reg = n_pi * iv_pi * iv_pi - 2.0 * b * iv_pi * iv_ne + n_ne * iv_ne * iv_ne
    reg = jnp.sum(_softplus(neg_reg - pos_reg))

    out_ref[0, 0] = cf + COFF * reg


_loss = pl.pallas_call(
    _loss_body,
    out_shape=jax.ShapeDtypeStruct((1, 1), jnp.float32),
    in_specs=[pl.BlockSpec(memory_space=pltpu.VMEM)],
    out_specs=pl.BlockSpec(memory_space=pltpu.SMEM),
)


@jax.jit
def kernel(repr_x, pos_neg_pair_t):
    pnp_flat = pos_neg_pair_t.reshape(BATCH * 5)
    g = _gather5(pnp_flat, repr_x)
    g2 = g.reshape(5 * R, 128)
    return _loss(g2)[0, 0]
